# Initial kernel scaffold; baseline (speedup 1.0000x reference)
import jax
import jax.numpy as jnp
from jax.experimental import pallas as pl


def kernel(x, edge_index, edge_weight, W1, b1, W2, b2):
    raise NotImplementedError("write your pallas kernel here")


# speedup vs baseline: 3.6915x; 3.6915x over previous
import jax
import jax.numpy as jnp
from jax import lax
from jax.experimental import pallas as pl
from jax.experimental.pallas import tpu as pltpu
from jax.experimental.pallas import tpu_sc as plsc

N = 10000
M = 10000
E = 320000
D_IN = 128
HID = 128
OUT = 64
HH = HID // 2

NC = 2
NS = 16
NW = NC * NS
EPW = E // NW
K = 80
NCHUNK = EPW // K
RCHUNK = 400
DPAD = 10240
COLS = DPAD // NS


def _sc_seg_sum(seg_rows, ntab):
  mesh = plsc.VectorSubcoreMesh(core_axis_name="c", subcore_axis_name="s")
  ch_total = seg_rows // RCHUNK
  ch_iters = -(-ch_total // NS)
  d = HH

  out_type = (jax.ShapeDtypeStruct((NC, seg_rows, d), jnp.float32),) * ntab

  scratch = [
      pltpu.VMEM((K,), jnp.int32),
      pltpu.VMEM((K,), jnp.int32),
      pltpu.VMEM((K, d), jnp.float32),
      pltpu.VMEM((RCHUNK, d), jnp.float32),
      pltpu.VMEM((RCHUNK, d), jnp.float32),
      pltpu.VMEM_SHARED((seg_rows, d), jnp.float32),
      pltpu.SemaphoreType.DMA,
  ]

  def body(*refs):
    tbls = refs[:ntab]
    gidx, sidx = refs[ntab], refs[ntab + 1]
    acc_os = refs[ntab + 2:2 * ntab + 2]
    gi_v, si_v, rows_v, zbuf_v, dbuf_v, acc_sh, sem = refs[2 * ntab + 2:]
    c = lax.axis_index("c")
    s = lax.axis_index("s")
    base = (c * NS + s) * EPW
    zeros16 = jnp.zeros((16,), jnp.float32)

    def zrow(r, carry):
      for cc in range(d // 16):
        zbuf_v[r, pl.ds(cc * 16, 16)] = zeros16
      return carry
    lax.fori_loop(0, RCHUNK, zrow, 0)

    for tbl, acc_o in zip(tbls, acc_os):
      def zacc(i, carry):
        ch = s + i * NS

        @pl.when(ch < ch_total)
        def _():
          pltpu.sync_copy(zbuf_v, acc_sh.at[pl.ds(ch * RCHUNK, RCHUNK)])
        return carry
      lax.fori_loop(0, ch_iters, zacc, 0)

      plsc.subcore_barrier()

      def chunk(j, carry):
        off = base + j * K
        pltpu.sync_copy(gidx.at[pl.ds(off, K)], gi_v)
        pltpu.sync_copy(sidx.at[pl.ds(off, K)], si_v)
        pltpu.async_copy(tbl.at[gi_v], rows_v, sem).wait()
        pltpu.sync_copy(rows_v, acc_sh.at[si_v], add=True)
        return carry
      lax.fori_loop(0, NCHUNK, chunk, 0)

      plsc.subcore_barrier()

      def dump(i, carry):
        ch = s + i * NS

        @pl.when(ch < ch_total)
        def _():
          start = ch * RCHUNK
          pltpu.sync_copy(acc_sh.at[pl.ds(start, RCHUNK)], dbuf_v)
          pltpu.sync_copy(dbuf_v, acc_o.at[c, pl.ds(start, RCHUNK)])
        return carry
      lax.fori_loop(0, ch_iters, dump, 0)

  return pl.kernel(body, out_type=out_type, mesh=mesh,
                   scratch_types=tuple(scratch),
                   compiler_params=pltpu.CompilerParams(
                       use_tc_tiling_on_sc=False))


def _sc_degrees():
  mesh = plsc.VectorSubcoreMesh(core_axis_name="c", subcore_axis_name="s")
  out_type = (jax.ShapeDtypeStruct((NC, DPAD, 16), jnp.float32),) * 2
  scratch = [
      pltpu.VMEM((K,), jnp.int32),
      pltpu.VMEM((K,), jnp.int32),
      pltpu.VMEM((K, 16), jnp.float32),
      pltpu.VMEM((COLS, 16), jnp.float32),
      pltpu.VMEM_SHARED((DPAD, 16), jnp.float32),
      pltpu.VMEM_SHARED((DPAD, 16), jnp.float32),
  ]

  def body(gidx, sidx, de_o, dv_o, gi_v, si_v, ones_v, dstage_v,
           dege_sh, degv_sh):
    c = lax.axis_index("c")
    s = lax.axis_index("s")
    base = (c * NS + s) * EPW
    zeros16 = jnp.zeros((16,), jnp.float32)
    ones16 = jnp.ones((16,), jnp.float32)

    def fill(r, carry):
      ones_v[r, pl.ds(0, 16)] = ones16
      return carry
    lax.fori_loop(0, K, fill, 0)

    def fill0(r, carry):
      dstage_v[r, pl.ds(0, 16)] = zeros16
      return carry
    lax.fori_loop(0, COLS, fill0, 0)

    pltpu.sync_copy(dstage_v, dege_sh.at[pl.ds(s * COLS, COLS)])
    pltpu.sync_copy(dstage_v, degv_sh.at[pl.ds(s * COLS, COLS)])
    plsc.subcore_barrier()

    def chunk(j, carry):
      off = base + j * K
      pltpu.sync_copy(gidx.at[pl.ds(off, K)], gi_v)
      pltpu.sync_copy(sidx.at[pl.ds(off, K)], si_v)
      pltpu.sync_copy(ones_v, dege_sh.at[si_v], add=True)
      pltpu.sync_copy(ones_v, degv_sh.at[gi_v], add=True)
      return carry
    lax.fori_loop(0, NCHUNK, chunk, 0)

    plsc.subcore_barrier()
    for sh, out_ref in ((dege_sh, de_o), (degv_sh, dv_o)):
      pltpu.sync_copy(sh.at[pl.ds(s * COLS, COLS)], dstage_v)
      pltpu.sync_copy(dstage_v, out_ref.at[c, pl.ds(s * COLS, COLS)])

  return pl.kernel(body, out_type=out_type, mesh=mesh,
                   scratch_types=tuple(scratch),
                   compiler_params=pltpu.CompilerParams(
                       use_tc_tiling_on_sc=False))


_sc_pass2 = _sc_seg_sum(M, 2)
_sc_pass1 = _sc_seg_sum(M, 1)
_sc_deg = _sc_degrees()

_BS = 1000


def _mm_split(x, w):
  n, din = x.shape

  def body(x_ref, w_ref, o0_ref, o1_ref):
    t = jnp.dot(x_ref[...], w_ref[...], preferred_element_type=jnp.float32)
    o0_ref[...] = t[:, :HH]
    o1_ref[...] = t[:, HH:]

  shp = jax.ShapeDtypeStruct((n, HH), jnp.float32)
  return pl.pallas_call(
      body,
      grid=(n // _BS,),
      in_specs=[pl.BlockSpec((_BS, din), lambda i: (i, 0)),
                pl.BlockSpec((din, HID), lambda i: (0, 0))],
      out_specs=[pl.BlockSpec((_BS, HH), lambda i: (i, 0))] * 2,
      out_shape=(shp, shp),
  )(x, w)


def _deg_inv(de_p, dv_p):
  def body(de_ref, dv_ref, ie_ref, iv_ref):
    ie_ref[...] = 1.0 / jnp.maximum(de_ref[0] + de_ref[1], 1.0)
    iv_ref[...] = 1.0 / jnp.maximum(dv_ref[0] + dv_ref[1], 1.0)

  shp = jax.ShapeDtypeStruct((DPAD // 128, 128), jnp.float32)
  return pl.pallas_call(body, out_shape=(shp, shp))(
      de_p.reshape(NC, DPAD // 128, 128), dv_p.reshape(NC, DPAD // 128, 128))


def _comb2(a0, a1, inv_col):
  n = a0.shape[1]

  def body(a0_ref, a1_ref, i_ref, o0_ref, o1_ref):
    o0_ref[...] = (a0_ref[0] + a0_ref[1]) * i_ref[...]
    o1_ref[...] = (a1_ref[0] + a1_ref[1]) * i_ref[...]

  shp = jax.ShapeDtypeStruct((n, HH), jnp.float32)
  return pl.pallas_call(
      body,
      grid=(n // _BS,),
      in_specs=[pl.BlockSpec((NC, _BS, HH), lambda i: (0, i, 0)),
                pl.BlockSpec((NC, _BS, HH), lambda i: (0, i, 0)),
                pl.BlockSpec((_BS, 1), lambda i: (i, 0))],
      out_specs=[pl.BlockSpec((_BS, HH), lambda i: (i, 0))] * 2,
      out_shape=(shp, shp),
  )(a0, a1, inv_col)


def _elu_norm_mm(b0, b1, inv_col, bias_row, w):
  n = b0.shape[1]
  dout = w.shape[1]

  def body(b0_ref, b1_ref, i_ref, bias_ref, w_ref, o_ref):
    t0 = (b0_ref[0] + b0_ref[1]) * i_ref[...] + bias_ref[:, :HH]
    t1 = (b1_ref[0] + b1_ref[1]) * i_ref[...] + bias_ref[:, HH:]
    t = jnp.concatenate([t0, t1], axis=1)
    h = jnp.where(t > 0.0, t, jnp.exp(t) - 1.0)
    o_ref[...] = jnp.dot(h, w_ref[...], preferred_element_type=jnp.float32)

  return pl.pallas_call(
      body,
      grid=(n // _BS,),
      in_specs=[pl.BlockSpec((NC, _BS, HH), lambda i: (0, i, 0)),
                pl.BlockSpec((NC, _BS, HH), lambda i: (0, i, 0)),
                pl.BlockSpec((_BS, 1), lambda i: (i, 0)),
                pl.BlockSpec((1, HID), lambda i: (0, 0)),
                pl.BlockSpec((HID, dout), lambda i: (0, 0))],
      out_specs=pl.BlockSpec((_BS, dout), lambda i: (i, 0)),
      out_shape=jax.ShapeDtypeStruct((n, dout), jnp.float32),
  )(b0, b1, inv_col, bias_row, w)


def _comb_scale(acc, inv_col, d):
  n = acc.shape[1]

  def body(a_ref, i_ref, o_ref):
    o_ref[...] = (a_ref[0] + a_ref[1]) * i_ref[...]

  return pl.pallas_call(
      body,
      grid=(n // _BS,),
      in_specs=[pl.BlockSpec((NC, _BS, d), lambda i: (0, i, 0)),
                pl.BlockSpec((_BS, 1), lambda i: (i, 0))],
      out_specs=pl.BlockSpec((_BS, d), lambda i: (i, 0)),
      out_shape=jax.ShapeDtypeStruct((n, d), jnp.float32),
  )(acc, inv_col)


def _final(acc, inv_col, bias_row, d):
  n = acc.shape[1]

  def body(a_ref, i_ref, b_ref, o_ref):
    o_ref[...] = (a_ref[0] + a_ref[1]) * i_ref[...] + b_ref[...]

  return pl.pallas_call(
      body,
      grid=(n // _BS,),
      in_specs=[pl.BlockSpec((NC, _BS, d), lambda i: (0, i, 0)),
                pl.BlockSpec((_BS, 1), lambda i: (i, 0)),
                pl.BlockSpec((1, d), lambda i: (0, 0))],
      out_specs=pl.BlockSpec((_BS, d), lambda i: (i, 0)),
      out_shape=jax.ShapeDtypeStruct((n, d), jnp.float32),
  )(acc, inv_col, bias_row)


def kernel(x, edge_index, edge_weight, W1, b1, W2, b2):
  node_idx = edge_index[0]
  he_idx = edge_index[1]

  xv0, xv1 = _mm_split(x, W1)
  de_p, dv_p = _sc_deg(node_idx, he_idx)
  a0, a1 = _sc_pass2(xv0, xv1, node_idx, he_idx)
  inv_e2d, inv_v2d = _deg_inv(de_p[:, :, 0], dv_p[:, :, 0])
  inv_e = inv_e2d.reshape(DPAD)[:M].reshape(M, 1)
  inv_v = inv_v2d.reshape(DPAD)[:N].reshape(N, 1)

  xe10, xe11 = _comb2(a0, a1, inv_e)
  b0, b1_acc = _sc_pass2(xe10, xe11, he_idx, node_idx)
  xv2 = _elu_norm_mm(b0, b1_acc, inv_v, b1.reshape(1, HID), W2)
  (c_acc,) = _sc_pass1(xv2, node_idx, he_idx)
  xe2 = _comb_scale(c_acc, inv_e, OUT)
  (d_acc,) = _sc_pass1(xe2, he_idx, node_idx)
  return _final(d_acc, inv_v, b2.reshape(1, OUT), OUT)


# speedup vs baseline: 13.6058x; 3.6857x over previous
import jax
import jax.numpy as jnp
from jax import lax
from jax.experimental import pallas as pl
from jax.experimental.pallas import tpu as pltpu
from jax.experimental.pallas import tpu_sc as plsc

N = 10000
M = 10000
E = 320000
D_IN = 128
HID = 128
OUT = 64
HH = HID // 2

NC = 2
NS = 16
NW = NC * NS
EPW = E // NW
K = 80
NCHUNK = EPW // K
NBUF = 8
NROUND = -(-NCHUNK // NBUF)
RCHUNK = 200
DPAD = 10240
COLS = DPAD // NS


def _sc_seg_sum(seg_rows, ntab):
  mesh = plsc.VectorSubcoreMesh(core_axis_name="c", subcore_axis_name="s")
  ch_total = seg_rows // RCHUNK
  ch_iters = -(-ch_total // NS)
  d = HH

  out_type = (jax.ShapeDtypeStruct((NC, seg_rows, d), jnp.float32),) * ntab

  scratch = [
      pltpu.VMEM((NCHUNK, K), jnp.int32),
      pltpu.VMEM((NCHUNK, K), jnp.int32),
      [pltpu.VMEM((K, d), jnp.float32) for _ in range(NBUF)],
      pltpu.VMEM((RCHUNK, d), jnp.float32),
      pltpu.VMEM_SHARED((seg_rows, d), jnp.float32),
      pltpu.SemaphoreType.DMA,
  ]

  def body(*refs):
    tbls = refs[:ntab]
    gidx3, sidx3 = refs[ntab], refs[ntab + 1]
    acc_os = refs[ntab + 2:2 * ntab + 2]
    gi_all, si_all, bufs, zbuf_v, acc_sh, sem = refs[2 * ntab + 2:]
    c = lax.axis_index("c")
    s = lax.axis_index("s")
    wid = c * NS + s
    zeros16 = jnp.zeros((16,), jnp.float32)

    pltpu.sync_copy(gidx3.at[wid], gi_all)
    pltpu.sync_copy(sidx3.at[wid], si_all)

    def zrow(r, carry):
      for cc in range(d // 16):
        zbuf_v[r, pl.ds(cc * 16, 16)] = zeros16
      return carry
    lax.fori_loop(0, RCHUNK, zrow, 0)

    for tbl, acc_o in zip(tbls, acc_os):
      def zacc(i, carry):
        ch = s + i * NS

        @pl.when(ch < ch_total)
        def _():
          pltpu.sync_copy(zbuf_v, acc_sh.at[pl.ds(ch * RCHUNK, RCHUNK)])
        return carry
      lax.fori_loop(0, ch_iters, zacc, 0)

      plsc.subcore_barrier()

      for b in range(NBUF):
        pltpu.async_copy(tbl.at[gi_all.at[b]], bufs[b], sem)

      def round_(r, carry):
        for b in range(NBUF):
          j = r * NBUF + b

          @pl.when(j < NCHUNK)
          def _():
            pltpu.make_async_copy(tbl.at[gi_all.at[j]], bufs[b], sem).wait()
            pltpu.sync_copy(bufs[b], acc_sh.at[si_all.at[j]], add=True)

            @pl.when(j + NBUF < NCHUNK)
            def _():
              pltpu.async_copy(tbl.at[gi_all.at[j + NBUF]], bufs[b], sem)
        return carry
      lax.fori_loop(0, NROUND, round_, 0)

      plsc.subcore_barrier()

      def dump(i, carry):
        ch = s + i * NS

        @pl.when(ch < ch_total)
        def _():
          start = ch * RCHUNK
          pltpu.sync_copy(acc_sh.at[pl.ds(start, RCHUNK)], zbuf_v)
          pltpu.sync_copy(zbuf_v, acc_o.at[c, pl.ds(start, RCHUNK)])
        return carry
      lax.fori_loop(0, ch_iters, dump, 0)

      lax.fori_loop(0, RCHUNK, zrow, 0)

  return pl.kernel(body, out_type=out_type, mesh=mesh,
                   scratch_types=tuple(scratch),
                   compiler_params=pltpu.CompilerParams(
                       use_tc_tiling_on_sc=False))


def _sc_degrees():
  mesh = plsc.VectorSubcoreMesh(core_axis_name="c", subcore_axis_name="s")
  out_type = (jax.ShapeDtypeStruct((NC, DPAD, 16), jnp.float32),) * 2
  scratch = [
      pltpu.VMEM((NCHUNK, K), jnp.int32),
      pltpu.VMEM((NCHUNK, K), jnp.int32),
      pltpu.VMEM((K, 16), jnp.float32),
      pltpu.VMEM((COLS, 16), jnp.float32),
      pltpu.VMEM_SHARED((DPAD, 16), jnp.float32),
      pltpu.VMEM_SHARED((DPAD, 16), jnp.float32),
      pltpu.SemaphoreType.DMA,
  ]

  def body(gidx3, sidx3, de_o, dv_o, gi_all, si_all, ones_v, dstage_v,
           dege_sh, degv_sh, sem):
    c = lax.axis_index("c")
    s = lax.axis_index("s")
    wid = c * NS + s
    zeros16 = jnp.zeros((16,), jnp.float32)
    ones16 = jnp.ones((16,), jnp.float32)

    pltpu.sync_copy(gidx3.at[wid], gi_all)
    pltpu.sync_copy(sidx3.at[wid], si_all)

    def fill(r, carry):
      ones_v[r, pl.ds(0, 16)] = ones16
      return carry
    lax.fori_loop(0, K, fill, 0)

    def fill0(r, carry):
      dstage_v[r, pl.ds(0, 16)] = zeros16
      return carry
    lax.fori_loop(0, COLS, fill0, 0)

    pltpu.sync_copy(dstage_v, dege_sh.at[pl.ds(s * COLS, COLS)])
    pltpu.sync_copy(dstage_v, degv_sh.at[pl.ds(s * COLS, COLS)])
    plsc.subcore_barrier()

    def chunk(j, carry):
      pltpu.async_copy(ones_v, dege_sh.at[si_all.at[j]], sem, add=True)
      pltpu.async_copy(ones_v, degv_sh.at[gi_all.at[j]], sem, add=True)
      return carry
    lax.fori_loop(0, NCHUNK, chunk, 0)

    def drain(j, carry):
      pltpu.make_async_copy(ones_v, dege_sh.at[si_all.at[0]], sem).wait()
      pltpu.make_async_copy(ones_v, degv_sh.at[gi_all.at[0]], sem).wait()
      return carry
    lax.fori_loop(0, NCHUNK, drain, 0)

    plsc.subcore_barrier()
    for sh, out_ref in ((dege_sh, de_o), (degv_sh, dv_o)):
      pltpu.sync_copy(sh.at[pl.ds(s * COLS, COLS)], dstage_v)
      pltpu.sync_copy(dstage_v, out_ref.at[c, pl.ds(s * COLS, COLS)])

  return pl.kernel(body, out_type=out_type, mesh=mesh,
                   scratch_types=tuple(scratch),
                   compiler_params=pltpu.CompilerParams(
                       use_tc_tiling_on_sc=False))


_sc_pass2 = _sc_seg_sum(M, 2)
_sc_pass1 = _sc_seg_sum(M, 1)
_sc_deg = _sc_degrees()

_BS = 1000


def _mm_split(x, w):
  n, din = x.shape

  def body(x_ref, w_ref, o0_ref, o1_ref):
    t = jnp.dot(x_ref[...], w_ref[...], preferred_element_type=jnp.float32)
    o0_ref[...] = t[:, :HH]
    o1_ref[...] = t[:, HH:]

  shp = jax.ShapeDtypeStruct((n, HH), jnp.float32)
  return pl.pallas_call(
      body,
      grid=(n // _BS,),
      in_specs=[pl.BlockSpec((_BS, din), lambda i: (i, 0)),
                pl.BlockSpec((din, HID), lambda i: (0, 0))],
      out_specs=[pl.BlockSpec((_BS, HH), lambda i: (i, 0))] * 2,
      out_shape=(shp, shp),
  )(x, w)


def _deg_inv(de_p, dv_p):
  def body(de_ref, dv_ref, ie_ref, iv_ref):
    ie_ref[...] = 1.0 / jnp.maximum(de_ref[0] + de_ref[1], 1.0)
    iv_ref[...] = 1.0 / jnp.maximum(dv_ref[0] + dv_ref[1], 1.0)

  shp = jax.ShapeDtypeStruct((DPAD // 128, 128), jnp.float32)
  return pl.pallas_call(body, out_shape=(shp, shp))(
      de_p.reshape(NC, DPAD // 128, 128), dv_p.reshape(NC, DPAD // 128, 128))


def _comb2(a0, a1, inv_col):
  n = a0.shape[1]

  def body(a0_ref, a1_ref, i_ref, o0_ref, o1_ref):
    o0_ref[...] = (a0_ref[0] + a0_ref[1]) * i_ref[...]
    o1_ref[...] = (a1_ref[0] + a1_ref[1]) * i_ref[...]

  shp = jax.ShapeDtypeStruct((n, HH), jnp.float32)
  return pl.pallas_call(
      body,
      grid=(n // _BS,),
      in_specs=[pl.BlockSpec((NC, _BS, HH), lambda i: (0, i, 0)),
                pl.BlockSpec((NC, _BS, HH), lambda i: (0, i, 0)),
                pl.BlockSpec((_BS, 1), lambda i: (i, 0))],
      out_specs=[pl.BlockSpec((_BS, HH), lambda i: (i, 0))] * 2,
      out_shape=(shp, shp),
  )(a0, a1, inv_col)


def _elu_norm_mm(b0, b1, inv_col, bias_row, w):
  n = b0.shape[1]
  dout = w.shape[1]

  def body(b0_ref, b1_ref, i_ref, bias_ref, w_ref, o_ref):
    t0 = (b0_ref[0] + b0_ref[1]) * i_ref[...] + bias_ref[:, :HH]
    t1 = (b1_ref[0] + b1_ref[1]) * i_ref[...] + bias_ref[:, HH:]
    t = jnp.concatenate([t0, t1], axis=1)
    h = jnp.where(t > 0.0, t, jnp.exp(t) - 1.0)
    o_ref[...] = jnp.dot(h, w_ref[...], preferred_element_type=jnp.float32)

  return pl.pallas_call(
      body,
      grid=(n // _BS,),
      in_specs=[pl.BlockSpec((NC, _BS, HH), lambda i: (0, i, 0)),
                pl.BlockSpec((NC, _BS, HH), lambda i: (0, i, 0)),
                pl.BlockSpec((_BS, 1), lambda i: (i, 0)),
                pl.BlockSpec((1, HID), lambda i: (0, 0)),
                pl.BlockSpec((HID, dout), lambda i: (0, 0))],
      out_specs=pl.BlockSpec((_BS, dout), lambda i: (i, 0)),
      out_shape=jax.ShapeDtypeStruct((n, dout), jnp.float32),
  )(b0, b1, inv_col, bias_row, w)


def _comb_scale(acc, inv_col, d):
  n = acc.shape[1]

  def body(a_ref, i_ref, o_ref):
    o_ref[...] = (a_ref[0] + a_ref[1]) * i_ref[...]

  return pl.pallas_call(
      body,
      grid=(n // _BS,),
      in_specs=[pl.BlockSpec((NC, _BS, d), lambda i: (0, i, 0)),
                pl.BlockSpec((_BS, 1), lambda i: (i, 0))],
      out_specs=pl.BlockSpec((_BS, d), lambda i: (i, 0)),
      out_shape=jax.ShapeDtypeStruct((n, d), jnp.float32),
  )(acc, inv_col)


def _final(acc, inv_col, bias_row, d):
  n = acc.shape[1]

  def body(a_ref, i_ref, b_ref, o_ref):
    o_ref[...] = (a_ref[0] + a_ref[1]) * i_ref[...] + b_ref[...]

  return pl.pallas_call(
      body,
      grid=(n // _BS,),
      in_specs=[pl.BlockSpec((NC, _BS, d), lambda i: (0, i, 0)),
                pl.BlockSpec((_BS, 1), lambda i: (i, 0)),
                pl.BlockSpec((1, d), lambda i: (0, 0))],
      out_specs=pl.BlockSpec((_BS, d), lambda i: (i, 0)),
      out_shape=jax.ShapeDtypeStruct((n, d), jnp.float32),
  )(acc, inv_col, bias_row)


def kernel(x, edge_index, edge_weight, W1, b1, W2, b2):
  node3 = edge_index[0].reshape(NW, NCHUNK, K)
  he3 = edge_index[1].reshape(NW, NCHUNK, K)

  xv0, xv1 = _mm_split(x, W1)
  de_p, dv_p = _sc_deg(node3, he3)
  a0, a1 = _sc_pass2(xv0, xv1, node3, he3)
  inv_e2d, inv_v2d = _deg_inv(de_p[:, :, 0], dv_p[:, :, 0])
  inv_e = inv_e2d.reshape(DPAD)[:M].reshape(M, 1)
  inv_v = inv_v2d.reshape(DPAD)[:N].reshape(N, 1)

  xe10, xe11 = _comb2(a0, a1, inv_e)
  b0, b1_acc = _sc_pass2(xe10, xe11, he3, node3)
  xv2 = _elu_norm_mm(b0, b1_acc, inv_v, b1.reshape(1, HID), W2)
  (c_acc,) = _sc_pass1(xv2, node3, he3)
  xe2 = _comb_scale(c_acc, inv_e, OUT)
  (d_acc,) = _sc_pass1(xe2, he3, node3)
  return _final(d_acc, inv_v, b2.reshape(1, OUT), OUT)
